# baseline (speedup 1.0000x reference)
import functools

import jax
import jax.numpy as jnp
from jax import lax
from jax.experimental import pallas as pl
from jax.experimental.pallas import tpu as pltpu
from jax.experimental.pallas import tpu_sc as plsc

N_ROWS = 32768
N_COLS = 256
N_SEG = 16
LANES = 16
NC, NS = 2, 16
NW = NC * NS

TC_ROWS = 16384
SC_ROWS = N_ROWS - TC_ROWS
ROWS_W = SC_ROWS // NW
CHUNK = 128
N_CHUNK = ROWS_W // CHUNK
GROUP = 16
N_GROUP = CHUNK // GROUP
CBLK = N_COLS // LANES

BR = 2048

_mesh = plsc.VectorSubcoreMesh(
    core_axis_name="c", subcore_axis_name="s", num_cores=NC, num_subcores=NS
)


@functools.partial(
    pl.kernel,
    out_type=jax.ShapeDtypeStruct((NW, N_SEG, N_COLS), jnp.float32),
    mesh=_mesh,
    compiler_params=pltpu.CompilerParams(use_tc_tiling_on_sc=True),
    scratch_types=[
        pltpu.VMEM((CHUNK, N_COLS), jnp.float32),
        pltpu.VMEM((CHUNK, N_COLS), jnp.float32),
        pltpu.VMEM((ROWS_W,), jnp.int32),
        pltpu.VMEM((N_SEG, N_COLS), jnp.float32),
        pltpu.SemaphoreType.DMA,
        pltpu.SemaphoreType.DMA,
    ],
)
def _sc_partial_max(x_hbm, ids_hbm, out_hbm, xbufa, xbufb, idsbuf, acc, sema, semb):
    cid = lax.axis_index("c")
    sid = lax.axis_index("s")
    wid = sid * NC + cid
    base = TC_ROWS + wid * ROWS_W

    neg_inf = jnp.full((LANES,), -jnp.inf, jnp.float32)

    def init_body(j, carry):
        col = pl.multiple_of(j * LANES, LANES)
        for s in range(N_SEG):
            acc[s, pl.ds(col, LANES)] = neg_inf
        return carry

    lax.fori_loop(0, CBLK, init_body, 0)

    pltpu.sync_copy(ids_hbm.at[pl.ds(base, ROWS_W)], idsbuf)

    def _chunk_src(ci):
        row0 = base + ci * CHUNK
        return x_hbm.at[pl.ds(pl.multiple_of(row0, CHUNK), CHUNK), :]

    def _start(ci, buf, sem):
        pltpu.make_async_copy(_chunk_src(ci), buf, sem).start()

    def _wait(buf, sem):
        pltpu.make_async_copy(_chunk_src(0), buf, sem).wait()

    def _process(ci, xbuf):
        def group_body(g, inner):
            lrow = pl.multiple_of(g * GROUP, GROUP)
            id0 = ci * CHUNK + lrow
            idv = idsbuf[pl.ds(pl.multiple_of(id0, LANES), GROUP)]
            lo = idv[0]
            hi = idv[GROUP - 1]

            @pl.when(lo == hi)
            def _fast():
                for j in range(CBLK):
                    col = j * LANES
                    m = acc[lo, pl.ds(col, LANES)]
                    for r in range(GROUP):
                        m = jnp.maximum(m, xbuf[lrow + r, pl.ds(col, LANES)])
                    acc[lo, pl.ds(col, LANES)] = m

            @pl.when(lo != hi)
            def _slow():
                for r in range(GROUP):
                    seg = idv[r]
                    for j in range(CBLK):
                        col = j * LANES
                        a = acc[seg, pl.ds(col, LANES)]
                        v = xbuf[lrow + r, pl.ds(col, LANES)]
                        acc[seg, pl.ds(col, LANES)] = jnp.maximum(a, v)

            return inner

        lax.fori_loop(0, N_GROUP, group_body, 0)

    _start(0, xbufa, sema)

    def outer_body(i, carry):
        ci = pl.multiple_of(i * 2, 2)
        _start(ci + 1, xbufb, semb)
        _wait(xbufa, sema)
        _process(ci, xbufa)

        @pl.when(ci + 2 < N_CHUNK)
        def _prefetch_a():
            _start(ci + 2, xbufa, sema)

        _wait(xbufb, semb)
        _process(ci + 1, xbufb)
        return carry

    lax.fori_loop(0, N_CHUNK // 2, outer_body, 0)

    pltpu.sync_copy(acc, out_hbm.at[wid])


def _tc_partial_body(b_ref, x_ref, o_ref):
    i = pl.program_id(0)

    @pl.when(i == 0)
    def _init():
        o_ref[:] = jnp.full((N_SEG, N_COLS), -jnp.inf, jnp.float32)

    xblk = x_ref[:]
    rowid = lax.broadcasted_iota(jnp.int32, (BR, 1), 0)
    blk0 = i * BR
    for s in range(N_SEG):
        r0 = jnp.maximum(b_ref[s] - blk0, 0)
        r1 = jnp.minimum(b_ref[s + 1] - blk0, BR)

        @pl.when(r1 > r0)
        def _update(s=s, r0=r0, r1=r1):
            mask = (rowid >= r0) & (rowid < r1)
            m = jnp.max(
                jnp.where(mask, xblk, -jnp.inf), axis=0, keepdims=True
            )
            o_ref[pl.ds(s, 1), :] = jnp.maximum(o_ref[pl.ds(s, 1), :], m)


_tc_partial = pl.pallas_call(
    _tc_partial_body,
    grid_spec=pltpu.PrefetchScalarGridSpec(
        num_scalar_prefetch=1,
        grid=(TC_ROWS // BR,),
        in_specs=[pl.BlockSpec((BR, N_COLS), lambda i, b: (i, 0))],
        out_specs=pl.BlockSpec((N_SEG, N_COLS), lambda i, b: (0, 0)),
    ),
    out_shape=jax.ShapeDtypeStruct((N_SEG, N_COLS), jnp.float32),
)


def _combine_body(p_ref, t_ref, o_ref):
    o_ref[:] = jnp.maximum(jnp.max(p_ref[:], axis=0), t_ref[:])


_combine = pl.pallas_call(
    _combine_body,
    out_shape=jax.ShapeDtypeStruct((N_SEG, N_COLS), jnp.float32),
)


@jax.jit
def kernel(x, batch_ids):
    ids = batch_ids.astype(jnp.int32)
    part = _sc_partial_max(x, ids)
    bounds = jnp.searchsorted(
        ids, jnp.arange(N_SEG + 1, dtype=jnp.int32), method="compare_all"
    ).astype(jnp.int32)
    tcp = _tc_partial(bounds, x)
    return _combine(part, tcp)


# speedup vs baseline: 1.0236x; 1.0236x over previous
import functools

import jax
import jax.numpy as jnp
from jax import lax
from jax.experimental import pallas as pl
from jax.experimental.pallas import tpu as pltpu
from jax.experimental.pallas import tpu_sc as plsc

N_ROWS = 32768
N_COLS = 256
N_SEG = 16
LANES = 16
NC, NS = 2, 16
NW = NC * NS

TC_ROWS = 16384
SC_ROWS = N_ROWS - TC_ROWS
ROWS_W = SC_ROWS // NW
CHUNK = 64
N_CHUNK = ROWS_W // CHUNK
GROUP = 16
N_GROUP = CHUNK // GROUP
CBLK = N_COLS // LANES

BR = 1024

_mesh = plsc.VectorSubcoreMesh(
    core_axis_name="c", subcore_axis_name="s", num_cores=NC, num_subcores=NS
)


@functools.partial(
    pl.kernel,
    out_type=jax.ShapeDtypeStruct((NW, N_SEG, N_COLS), jnp.float32),
    mesh=_mesh,
    compiler_params=pltpu.CompilerParams(use_tc_tiling_on_sc=True),
    scratch_types=[
        pltpu.VMEM((CHUNK, N_COLS), jnp.float32),
        pltpu.VMEM((CHUNK, N_COLS), jnp.float32),
        pltpu.VMEM((ROWS_W,), jnp.int32),
        pltpu.VMEM((N_SEG, N_COLS), jnp.float32),
        pltpu.SemaphoreType.DMA,
        pltpu.SemaphoreType.DMA,
    ],
)
def _sc_partial_max(x_hbm, ids_hbm, out_hbm, xbufa, xbufb, idsbuf, acc, sema, semb):
    cid = lax.axis_index("c")
    sid = lax.axis_index("s")
    wid = sid * NC + cid
    base = TC_ROWS + wid * ROWS_W

    neg_inf = jnp.full((LANES,), -jnp.inf, jnp.float32)

    def init_body(j, carry):
        col = pl.multiple_of(j * LANES, LANES)
        for s in range(N_SEG):
            acc[s, pl.ds(col, LANES)] = neg_inf
        return carry

    lax.fori_loop(0, CBLK, init_body, 0)

    pltpu.sync_copy(ids_hbm.at[pl.ds(base, ROWS_W)], idsbuf)

    def _chunk_src(ci):
        row0 = base + ci * CHUNK
        return x_hbm.at[pl.ds(pl.multiple_of(row0, CHUNK), CHUNK), :]

    def _start(ci, buf, sem):
        pltpu.make_async_copy(_chunk_src(ci), buf, sem).start()

    def _wait(buf, sem):
        pltpu.make_async_copy(_chunk_src(0), buf, sem).wait()

    def _process(ci, xbuf):
        def group_body(g, inner):
            lrow = pl.multiple_of(g * GROUP, GROUP)
            id0 = ci * CHUNK + lrow
            idv = idsbuf[pl.ds(pl.multiple_of(id0, LANES), GROUP)]
            lo = idv[0]
            hi = idv[GROUP - 1]

            @pl.when(lo == hi)
            def _fast():
                for j in range(CBLK):
                    col = j * LANES
                    m = acc[lo, pl.ds(col, LANES)]
                    for r in range(GROUP):
                        m = jnp.maximum(m, xbuf[lrow + r, pl.ds(col, LANES)])
                    acc[lo, pl.ds(col, LANES)] = m

            @pl.when(lo != hi)
            def _slow():
                for r in range(GROUP):
                    seg = idv[r]
                    for j in range(CBLK):
                        col = j * LANES
                        a = acc[seg, pl.ds(col, LANES)]
                        v = xbuf[lrow + r, pl.ds(col, LANES)]
                        acc[seg, pl.ds(col, LANES)] = jnp.maximum(a, v)

            return inner

        lax.fori_loop(0, N_GROUP, group_body, 0)

    _start(0, xbufa, sema)

    def outer_body(i, carry):
        ci = pl.multiple_of(i * 2, 2)
        _start(ci + 1, xbufb, semb)
        _wait(xbufa, sema)
        _process(ci, xbufa)

        @pl.when(ci + 2 < N_CHUNK)
        def _prefetch_a():
            _start(ci + 2, xbufa, sema)

        _wait(xbufb, semb)
        _process(ci + 1, xbufb)
        return carry

    lax.fori_loop(0, N_CHUNK // 2, outer_body, 0)

    pltpu.sync_copy(acc, out_hbm.at[wid])


def _tc_partial_body(b_ref, x_ref, o_ref):
    i = pl.program_id(0)

    @pl.when(i == 0)
    def _init():
        o_ref[:] = jnp.full((N_SEG, N_COLS), -jnp.inf, jnp.float32)

    xblk = x_ref[:]
    rowid = lax.broadcasted_iota(jnp.int32, (BR, 1), 0)
    blk0 = i * BR
    for s in range(N_SEG):
        r0 = jnp.maximum(b_ref[s] - blk0, 0)
        r1 = jnp.minimum(b_ref[s + 1] - blk0, BR)

        @pl.when(r1 > r0)
        def _update(s=s, r0=r0, r1=r1):
            mask = (rowid >= r0) & (rowid < r1)
            m = jnp.max(
                jnp.where(mask, xblk, -jnp.inf), axis=0, keepdims=True
            )
            o_ref[pl.ds(s, 1), :] = jnp.maximum(o_ref[pl.ds(s, 1), :], m)


_tc_partial = pl.pallas_call(
    _tc_partial_body,
    grid_spec=pltpu.PrefetchScalarGridSpec(
        num_scalar_prefetch=1,
        grid=(TC_ROWS // BR,),
        in_specs=[pl.BlockSpec((BR, N_COLS), lambda i, b: (i, 0))],
        out_specs=pl.BlockSpec((N_SEG, N_COLS), lambda i, b: (0, 0)),
    ),
    out_shape=jax.ShapeDtypeStruct((N_SEG, N_COLS), jnp.float32),
)


def _combine_body(p_ref, t_ref, o_ref):
    o_ref[:] = jnp.maximum(jnp.max(p_ref[:], axis=0), t_ref[:])


_combine = pl.pallas_call(
    _combine_body,
    out_shape=jax.ShapeDtypeStruct((N_SEG, N_COLS), jnp.float32),
)


@jax.jit
def kernel(x, batch_ids):
    ids = batch_ids.astype(jnp.int32)
    part = _sc_partial_max(x, ids)
    bounds = jnp.searchsorted(
        ids, jnp.arange(N_SEG + 1, dtype=jnp.int32), method="compare_all"
    ).astype(jnp.int32)
    tcp = _tc_partial(bounds, x)
    return _combine(part, tcp)


# speedup vs baseline: 1.0867x; 1.0616x over previous
import functools

import jax
import jax.numpy as jnp
from jax import lax
from jax.experimental import pallas as pl
from jax.experimental.pallas import tpu as pltpu
from jax.experimental.pallas import tpu_sc as plsc

N_ROWS = 32768
N_COLS = 256
N_SEG = 16
LANES = 16
NC, NS = 2, 16
NW = NC * NS

TC_ROWS = 16384
SC_ROWS = N_ROWS - TC_ROWS
ROWS_W = SC_ROWS // NW
CHUNK = 64
N_CHUNK = ROWS_W // CHUNK
GROUP = 16
N_GROUP = CHUNK // GROUP
CBLK = N_COLS // LANES

BR = 1024

_mesh = plsc.VectorSubcoreMesh(
    core_axis_name="c", subcore_axis_name="s", num_cores=NC, num_subcores=NS
)


@functools.partial(
    pl.kernel,
    out_type=jax.ShapeDtypeStruct((NW, N_SEG, N_COLS), jnp.float32),
    mesh=_mesh,
    compiler_params=pltpu.CompilerParams(use_tc_tiling_on_sc=True),
    scratch_types=[
        pltpu.VMEM((CHUNK, N_COLS), jnp.float32),
        pltpu.VMEM((CHUNK, N_COLS), jnp.float32),
        pltpu.VMEM((ROWS_W + LANES,), jnp.int32),
        pltpu.VMEM((N_SEG, N_COLS), jnp.float32),
        pltpu.SemaphoreType.DMA,
        pltpu.SemaphoreType.DMA,
    ],
)
def _sc_partial_max(x_hbm, ids_hbm, out_hbm, xbufa, xbufb, idsbuf, acc, sema, semb):
    cid = lax.axis_index("c")
    sid = lax.axis_index("s")
    wid = sid * NC + cid
    base = TC_ROWS + wid * ROWS_W

    neg_inf = jnp.full((LANES,), -jnp.inf, jnp.float32)

    def init_body(j, carry):
        col = pl.multiple_of(j * LANES, LANES)
        for s in range(N_SEG):
            acc[s, pl.ds(col, LANES)] = neg_inf
        return carry

    lax.fori_loop(0, CBLK, init_body, 0)

    pltpu.sync_copy(ids_hbm.at[pl.ds(base, ROWS_W)], idsbuf.at[pl.ds(0, ROWS_W)])

    def _chunk_src(ci):
        row0 = base + ci * CHUNK
        return x_hbm.at[pl.ds(pl.multiple_of(row0, CHUNK), CHUNK), :]

    def _start(ci, buf, sem):
        pltpu.make_async_copy(_chunk_src(ci), buf, sem).start()

    def _wait(buf, sem):
        pltpu.make_async_copy(_chunk_src(0), buf, sem).wait()

    def _process(ci, xbuf):
        def group_body(g, inner):
            lrow = pl.multiple_of(g * GROUP, GROUP)
            id0 = ci * CHUNK + lrow
            idv = idsbuf[pl.ds(pl.multiple_of(id0, LANES), GROUP)]
            lo = idv[0]
            hi = idv[GROUP - 1]

            @pl.when(lo == hi)
            def _fast():
                for j in range(CBLK):
                    col = j * LANES
                    m = acc[lo, pl.ds(col, LANES)]
                    for r in range(GROUP):
                        m = jnp.maximum(m, xbuf[lrow + r, pl.ds(col, LANES)])
                    acc[lo, pl.ds(col, LANES)] = m

            @pl.when(lo != hi)
            def _slow():
                def row_body(r, carry):
                    seg = idsbuf[pl.ds(id0 + r, LANES)][0]
                    for j in range(CBLK):
                        col = j * LANES
                        a = acc[seg, pl.ds(col, LANES)]
                        v = xbuf[lrow + r, pl.ds(col, LANES)]
                        acc[seg, pl.ds(col, LANES)] = jnp.maximum(a, v)
                    return carry

                lax.fori_loop(0, GROUP, row_body, 0)

            return inner

        lax.fori_loop(0, N_GROUP, group_body, 0)

    _start(0, xbufa, sema)

    def outer_body(i, carry):
        ci = pl.multiple_of(i * 2, 2)
        _start(ci + 1, xbufb, semb)
        _wait(xbufa, sema)
        _process(ci, xbufa)

        @pl.when(ci + 2 < N_CHUNK)
        def _prefetch_a():
            _start(ci + 2, xbufa, sema)

        _wait(xbufb, semb)
        _process(ci + 1, xbufb)
        return carry

    lax.fori_loop(0, N_CHUNK // 2, outer_body, 0)

    pltpu.sync_copy(acc, out_hbm.at[wid])


def _tc_partial_body(b_ref, x_ref, o_ref):
    i = pl.program_id(0)

    @pl.when(i == 0)
    def _init():
        o_ref[:] = jnp.full((N_SEG, N_COLS), -jnp.inf, jnp.float32)

    xblk = x_ref[:]
    rowid = lax.broadcasted_iota(jnp.int32, (BR, 1), 0)
    blk0 = i * BR
    for s in range(N_SEG):
        r0 = jnp.maximum(b_ref[s] - blk0, 0)
        r1 = jnp.minimum(b_ref[s + 1] - blk0, BR)

        @pl.when(r1 > r0)
        def _update(s=s, r0=r0, r1=r1):
            mask = (rowid >= r0) & (rowid < r1)
            m = jnp.max(
                jnp.where(mask, xblk, -jnp.inf), axis=0, keepdims=True
            )
            o_ref[pl.ds(s, 1), :] = jnp.maximum(o_ref[pl.ds(s, 1), :], m)


_tc_partial = pl.pallas_call(
    _tc_partial_body,
    grid_spec=pltpu.PrefetchScalarGridSpec(
        num_scalar_prefetch=1,
        grid=(TC_ROWS // BR,),
        in_specs=[pl.BlockSpec((BR, N_COLS), lambda i, b: (i, 0))],
        out_specs=pl.BlockSpec((N_SEG, N_COLS), lambda i, b: (0, 0)),
    ),
    out_shape=jax.ShapeDtypeStruct((N_SEG, N_COLS), jnp.float32),
)


def _combine_body(p_ref, t_ref, o_ref):
    o_ref[:] = jnp.maximum(jnp.max(p_ref[:], axis=0), t_ref[:])


_combine = pl.pallas_call(
    _combine_body,
    out_shape=jax.ShapeDtypeStruct((N_SEG, N_COLS), jnp.float32),
)


@jax.jit
def kernel(x, batch_ids):
    ids = batch_ids.astype(jnp.int32)
    part = _sc_partial_max(x, ids)
    bounds = jnp.searchsorted(
        ids, jnp.arange(N_SEG + 1, dtype=jnp.int32), method="compare_all"
    ).astype(jnp.int32)
    tcp = _tc_partial(bounds, x)
    return _combine(part, tcp)


# speedup vs baseline: 1.0879x; 1.0011x over previous
import functools

import jax
import jax.numpy as jnp
from jax import lax
from jax.experimental import pallas as pl
from jax.experimental.pallas import tpu as pltpu
from jax.experimental.pallas import tpu_sc as plsc

N_ROWS = 32768
N_COLS = 256
N_SEG = 16
LANES = 16
NC, NS = 2, 16
NW = NC * NS

TC_ROWS = 16384
SC_ROWS = N_ROWS - TC_ROWS
ROWS_W = SC_ROWS // NW
CHUNK = 64
N_CHUNK = ROWS_W // CHUNK
GROUP = 16
N_GROUP = CHUNK // GROUP
CBLK = N_COLS // LANES

BR = 1024

_mesh = plsc.VectorSubcoreMesh(
    core_axis_name="c", subcore_axis_name="s", num_cores=NC, num_subcores=NS
)


@functools.partial(
    pl.kernel,
    out_type=jax.ShapeDtypeStruct((NW, N_SEG, N_COLS), jnp.float32),
    mesh=_mesh,
    compiler_params=pltpu.CompilerParams(use_tc_tiling_on_sc=True),
    scratch_types=[
        pltpu.VMEM((CHUNK, N_COLS), jnp.float32),
        pltpu.VMEM((CHUNK, N_COLS), jnp.float32),
        pltpu.VMEM((ROWS_W + LANES,), jnp.int32),
        pltpu.VMEM((N_SEG, N_COLS), jnp.float32),
        pltpu.SemaphoreType.DMA,
        pltpu.SemaphoreType.DMA,
    ],
)
def _sc_partial_max(x_hbm, ids_hbm, out_hbm, xbufa, xbufb, idsbuf, acc, sema, semb):
    cid = lax.axis_index("c")
    sid = lax.axis_index("s")
    wid = sid * NC + cid
    base = TC_ROWS + wid * ROWS_W

    neg_inf = jnp.full((LANES,), -jnp.inf, jnp.float32)

    def init_body(j, carry):
        col = pl.multiple_of(j * LANES, LANES)
        for s in range(N_SEG):
            acc[s, pl.ds(col, LANES)] = neg_inf
        return carry

    lax.fori_loop(0, CBLK, init_body, 0)

    pltpu.sync_copy(ids_hbm.at[pl.ds(base, ROWS_W)], idsbuf.at[pl.ds(0, ROWS_W)])

    def _chunk_src(ci):
        row0 = base + ci * CHUNK
        return x_hbm.at[pl.ds(pl.multiple_of(row0, CHUNK), CHUNK), :]

    def _start(ci, buf, sem):
        pltpu.make_async_copy(_chunk_src(ci), buf, sem).start()

    def _wait(buf, sem):
        pltpu.make_async_copy(_chunk_src(0), buf, sem).wait()

    def _process(ci, xbuf):
        def group_body(g, inner):
            lrow = pl.multiple_of(g * GROUP, GROUP)
            id0 = ci * CHUNK + lrow
            idv = idsbuf[pl.ds(pl.multiple_of(id0, LANES), GROUP)]
            lo = idv[0]
            hi = idv[GROUP - 1]

            @pl.when(lo == hi)
            def _fast():
                def col_body(j, carry):
                    col = pl.multiple_of(j * LANES, LANES)
                    m = acc[lo, pl.ds(col, LANES)]
                    for r in range(GROUP):
                        m = jnp.maximum(m, xbuf[lrow + r, pl.ds(col, LANES)])
                    acc[lo, pl.ds(col, LANES)] = m
                    return carry

                lax.fori_loop(0, CBLK, col_body, 0)

            @pl.when(lo != hi)
            def _slow():
                def row_body(r, carry):
                    seg = idsbuf[pl.ds(id0 + r, LANES)][0]

                    def rcol_body(j, inner):
                        col = pl.multiple_of(j * LANES, LANES)
                        a = acc[seg, pl.ds(col, LANES)]
                        v = xbuf[lrow + r, pl.ds(col, LANES)]
                        acc[seg, pl.ds(col, LANES)] = jnp.maximum(a, v)
                        return inner

                    lax.fori_loop(0, CBLK, rcol_body, 0)
                    return carry

                lax.fori_loop(0, GROUP, row_body, 0)

            return inner

        lax.fori_loop(0, N_GROUP, group_body, 0)

    _start(0, xbufa, sema)

    def outer_body(i, carry):
        ci = pl.multiple_of(i * 2, 2)
        _start(ci + 1, xbufb, semb)
        _wait(xbufa, sema)
        _process(ci, xbufa)

        @pl.when(ci + 2 < N_CHUNK)
        def _prefetch_a():
            _start(ci + 2, xbufa, sema)

        _wait(xbufb, semb)
        _process(ci + 1, xbufb)
        return carry

    lax.fori_loop(0, N_CHUNK // 2, outer_body, 0)

    pltpu.sync_copy(acc, out_hbm.at[wid])


def _tc_partial_body(b_ref, x_ref, o_ref):
    i = pl.program_id(0)

    @pl.when(i == 0)
    def _init():
        o_ref[:] = jnp.full((N_SEG, N_COLS), -jnp.inf, jnp.float32)

    xblk = x_ref[:]
    rowid = lax.broadcasted_iota(jnp.int32, (BR, 1), 0)
    blk0 = i * BR
    for s in range(N_SEG):
        r0 = jnp.maximum(b_ref[s] - blk0, 0)
        r1 = jnp.minimum(b_ref[s + 1] - blk0, BR)

        @pl.when(r1 > r0)
        def _update(s=s, r0=r0, r1=r1):
            mask = (rowid >= r0) & (rowid < r1)
            m = jnp.max(
                jnp.where(mask, xblk, -jnp.inf), axis=0, keepdims=True
            )
            o_ref[pl.ds(s, 1), :] = jnp.maximum(o_ref[pl.ds(s, 1), :], m)


_tc_partial = pl.pallas_call(
    _tc_partial_body,
    grid_spec=pltpu.PrefetchScalarGridSpec(
        num_scalar_prefetch=1,
        grid=(TC_ROWS // BR,),
        in_specs=[pl.BlockSpec((BR, N_COLS), lambda i, b: (i, 0))],
        out_specs=pl.BlockSpec((N_SEG, N_COLS), lambda i, b: (0, 0)),
    ),
    out_shape=jax.ShapeDtypeStruct((N_SEG, N_COLS), jnp.float32),
)


def _combine_body(p_ref, t_ref, o_ref):
    o_ref[:] = jnp.maximum(jnp.max(p_ref[:], axis=0), t_ref[:])


_combine = pl.pallas_call(
    _combine_body,
    out_shape=jax.ShapeDtypeStruct((N_SEG, N_COLS), jnp.float32),
)


@jax.jit
def kernel(x, batch_ids):
    ids = batch_ids.astype(jnp.int32)
    part = _sc_partial_max(x, ids)
    bounds = jnp.searchsorted(
        ids, jnp.arange(N_SEG + 1, dtype=jnp.int32), method="compare_all"
    ).astype(jnp.int32)
    tcp = _tc_partial(bounds, x)
    return _combine(part, tcp)
